# Initial kernel scaffold; baseline (speedup 1.0000x reference)
import jax
import jax.numpy as jnp
from jax.experimental import pallas as pl


def kernel(predict, target):
    raise NotImplementedError("write your pallas kernel here")


# speedup vs baseline: 11.3068x; 11.3068x over previous
import functools

import jax
import jax.numpy as jnp
from jax import lax
from jax.experimental import pallas as pl
from jax.experimental.pallas import tpu as pltpu
from jax.experimental.pallas import tpu_sc as plsc

_IGNORE = 255
_RATIO = 0.7

_N, _C, _H, _W = 8, 19, 512, 512
_NPIX = _N * _H * _W
_RB = 32

_NW = 32
_PER_TILE = _NPIX // _NW
_SLAB = 32768
_B1 = 65536
_B2 = 32768


def _probs_body(x_ref, t_ref, p_ref, n_ref):
    x = x_ref[...]
    tgt = t_ref[...]
    m = jnp.max(x, axis=1)
    se = jnp.sum(jnp.exp(x - m[:, None]), axis=1)
    cls = lax.broadcasted_iota(jnp.int32, x.shape, 1)
    xt = jnp.sum(jnp.where(cls == tgt[:, None], x, 0.0), axis=1)
    pt = jnp.exp(xt - m) / se
    valid = tgt != _IGNORE
    p_ref[...] = lax.bitcast_convert_type(
        jnp.where(valid, pt, -1.0), jnp.int32
    )
    n_ref[...] = jnp.log(se) - (xt - m)


def _probs_call(predict, target):
    grid = (_N, _H // _RB)
    return pl.pallas_call(
        _probs_body,
        grid=grid,
        in_specs=[
            pl.BlockSpec((1, _C, _RB, _W), lambda b, r: (b, 0, r, 0)),
            pl.BlockSpec((1, _RB, _W), lambda b, r: (b, r, 0)),
        ],
        out_specs=[
            pl.BlockSpec((1, _RB, _W), lambda b, r: (b, r, 0)),
            pl.BlockSpec((1, _RB, _W), lambda b, r: (b, r, 0)),
        ],
        out_shape=[
            jax.ShapeDtypeStruct((_N, _H, _W), jnp.int32),
            jax.ShapeDtypeStruct((_N, _H, _W), jnp.float32),
        ],
    )(predict, target)


@functools.partial(
    pl.kernel,
    mesh=plsc.VectorSubcoreMesh(core_axis_name="c", subcore_axis_name="s"),
    out_type=jax.ShapeDtypeStruct((_NW, _B1), jnp.int32),
    scratch_types=[
        pltpu.VMEM((_SLAB,), jnp.int32),
        pltpu.VMEM((_B1,), jnp.int32),
    ],
    compiler_params=pltpu.CompilerParams(needs_layout_passes=False),
)
def _hist_pass_a(pred_hbm, out_hbm, buf, hist):
    wid = lax.axis_index("s") * 2 + lax.axis_index("c")
    base = wid * _PER_TILE

    def zero_body(i, c):
        hist[pl.ds(i * 16, 16)] = jnp.zeros((16,), jnp.int32)
        return c

    lax.fori_loop(0, _B1 // 16, zero_body, 0)
    ones = jnp.ones((16,), jnp.int32)

    def slab_body(s, c):
        pltpu.sync_copy(pred_hbm.at[pl.ds(base + s * _SLAB, _SLAB)], buf)

        def elem_body(i, c2):
            bits = buf[pl.ds(i * 16, 16)]
            ok = bits >= 0
            b = jnp.where(ok, bits >> 15, 0)
            plsc.addupdate_scatter(hist, [b], ones, mask=ok)
            return c2

        lax.fori_loop(0, _SLAB // 16, elem_body, 0)
        return c

    lax.fori_loop(0, _PER_TILE // _SLAB, slab_body, 0)
    pltpu.sync_copy(hist, out_hbm.at[wid])


@functools.partial(
    pl.kernel,
    mesh=plsc.VectorSubcoreMesh(core_axis_name="c", subcore_axis_name="s"),
    out_type=jax.ShapeDtypeStruct((_NW, _B2), jnp.int32),
    scratch_types=[
        pltpu.VMEM((_SLAB,), jnp.int32),
        pltpu.VMEM((_B2,), jnp.int32),
        pltpu.VMEM((16,), jnp.int32),
    ],
    compiler_params=pltpu.CompilerParams(needs_layout_passes=False),
)
def _hist_pass_b(pred_hbm, b1_hbm, out_hbm, buf, hist, b1buf):
    wid = lax.axis_index("s") * 2 + lax.axis_index("c")
    base = wid * _PER_TILE
    pltpu.sync_copy(b1_hbm, b1buf)
    b1 = b1buf[...]

    def zero_body(i, c):
        hist[pl.ds(i * 16, 16)] = jnp.zeros((16,), jnp.int32)
        return c

    lax.fori_loop(0, _B2 // 16, zero_body, 0)
    ones = jnp.ones((16,), jnp.int32)

    def slab_body(s, c):
        pltpu.sync_copy(pred_hbm.at[pl.ds(base + s * _SLAB, _SLAB)], buf)

        def elem_body(i, c2):
            bits = buf[pl.ds(i * 16, 16)]
            ok = (bits >= 0) & ((bits >> 15) == b1)
            b = jnp.where(ok, bits & 0x7FFF, 0)
            plsc.addupdate_scatter(hist, [b], ones, mask=ok)
            return c2

        lax.fori_loop(0, _SLAB // 16, elem_body, 0)
        return c

    lax.fori_loop(0, _PER_TILE // _SLAB, slab_body, 0)
    pltpu.sync_copy(hist, out_hbm.at[wid])


def _search(h, binidx, k, nbins, iters):
    def gcount(m):
        return jnp.sum(jnp.where(binidx > m, h, 0))

    def body(_, lohi):
        lo, hi = lohi
        mid = (lo + hi) // 2
        below = gcount(mid) < k
        return (jnp.where(below, lo, mid), jnp.where(below, mid, hi))

    lo, hi = lax.fori_loop(
        0, iters, body, (jnp.int32(-1), jnp.int32(nbins - 1))
    )
    return hi, gcount(hi)


def _scan_a_body(h_ref, b1_ref, kp_ref):
    h = jnp.sum(h_ref[...], axis=0)
    r = lax.broadcasted_iota(jnp.int32, h.shape, 0)
    c = lax.broadcasted_iota(jnp.int32, h.shape, 1)
    binidx = r * 128 + c
    nv = jnp.sum(h)
    k = jnp.floor(nv.astype(jnp.float32) * _RATIO).astype(jnp.int32)
    b1, above = _search(h, binidx, k, _B1, 18)
    b1_ref[...] = jnp.full((1, 128), b1, jnp.int32)
    kp_ref[...] = jnp.full((1, 128), k - above, jnp.int32)


def _scan_a_call(h1):
    return pl.pallas_call(
        _scan_a_body,
        out_shape=[
            jax.ShapeDtypeStruct((1, 128), jnp.int32),
            jax.ShapeDtypeStruct((1, 128), jnp.int32),
        ],
    )(h1)


def _scan_b_body(h_ref, b1_ref, kp_ref, t_ref):
    h = jnp.sum(h_ref[...], axis=0)
    r = lax.broadcasted_iota(jnp.int32, h.shape, 0)
    c = lax.broadcasted_iota(jnp.int32, h.shape, 1)
    binidx = r * 128 + c
    b1 = b1_ref[0, 0]
    kp = kp_ref[0, 0]
    b2, _ = _search(h, binidx, kp, _B2, 17)
    t_ref[...] = jnp.full((1, 128), (b1 << 15) | b2, jnp.int32)


def _scan_b_call(h2, b1v, kpv):
    return pl.pallas_call(
        _scan_b_body,
        in_specs=[
            pl.BlockSpec((_NW, _B2 // 128, 128), lambda: (0, 0, 0)),
            pl.BlockSpec(memory_space=pltpu.SMEM),
            pl.BlockSpec(memory_space=pltpu.SMEM),
        ],
        out_shape=jax.ShapeDtypeStruct((1, 128), jnp.int32),
    )(h2, b1v, kpv)


def _final_body(p_ref, n_ref, t_ref, out_ref, acc_ref):
    i = pl.program_id(0)
    j = pl.program_id(1)

    @pl.when((i == 0) & (j == 0))
    def _():
        acc_ref[0] = 0.0
        acc_ref[1] = 0.0

    p = p_ref[...]
    t = t_ref[0, 0]
    kept = (p >= 0) & (p <= t)
    acc_ref[0] += jnp.sum(jnp.where(kept, n_ref[...], 0.0))
    acc_ref[1] += jnp.sum(kept.astype(jnp.float32))

    @pl.when((i == _N - 1) & (j == _H // _RB - 1))
    def _():
        out_ref[...] = jnp.full(
            (1, 1), acc_ref[0] / jnp.maximum(acc_ref[1], 1.0), jnp.float32
        )


def _final_call(pred, nll, tv):
    grid = (_N, _H // _RB)
    return pl.pallas_call(
        _final_body,
        grid=grid,
        in_specs=[
            pl.BlockSpec((1, _RB, _W), lambda b, r: (b, r, 0)),
            pl.BlockSpec((1, _RB, _W), lambda b, r: (b, r, 0)),
            pl.BlockSpec(memory_space=pltpu.SMEM),
        ],
        out_specs=pl.BlockSpec((1, 1), lambda b, r: (0, 0)),
        out_shape=jax.ShapeDtypeStruct((1, 1), jnp.float32),
        scratch_shapes=[pltpu.SMEM((2,), jnp.float32)],
    )(pred, nll, tv)


def kernel(predict, target):
    pred, nll = _probs_call(predict, target)
    predf = pred.reshape(_NPIX)
    h1 = _hist_pass_a(predf)
    b1v, kpv = _scan_a_call(h1.reshape(_NW, _B1 // 128, 128))
    h2 = _hist_pass_b(predf, b1v[0, :16])
    tv = _scan_b_call(h2.reshape(_NW, _B2 // 128, 128), b1v, kpv)
    loss = _final_call(pred, nll, tv)
    return loss[0, 0]


# speedup vs baseline: 14.7995x; 1.3089x over previous
import functools

import jax
import jax.numpy as jnp
from jax import lax
from jax.experimental import pallas as pl
from jax.experimental.pallas import tpu as pltpu
from jax.experimental.pallas import tpu_sc as plsc

_IGNORE = 255
_RATIO = 0.7

_N, _C, _H, _W = 8, 19, 512, 512
_NPIX = _N * _H * _W
_RB = 32

_NW = 32
_PER_TILE = _NPIX // _NW
_SLAB = 32768
_B1 = 65536
_B2 = 32768


def _probs_body(x_ref, t_ref, p_ref, n_ref):
    x = x_ref[...]
    tgt = t_ref[...]
    m = jnp.max(x, axis=1)
    se = jnp.sum(jnp.exp(x - m[:, None]), axis=1)
    cls = lax.broadcasted_iota(jnp.int32, x.shape, 1)
    xt = jnp.sum(jnp.where(cls == tgt[:, None], x, 0.0), axis=1)
    pt = jnp.exp(xt - m) / se
    valid = tgt != _IGNORE
    p_ref[...] = lax.bitcast_convert_type(
        jnp.where(valid, pt, -1.0), jnp.int32
    )
    n_ref[...] = jnp.log(se) - (xt - m)


def _probs_call(predict, target):
    grid = (_N, _H // _RB)
    return pl.pallas_call(
        _probs_body,
        grid=grid,
        in_specs=[
            pl.BlockSpec((1, _C, _RB, _W), lambda b, r: (b, 0, r, 0)),
            pl.BlockSpec((1, _RB, _W), lambda b, r: (b, r, 0)),
        ],
        out_specs=[
            pl.BlockSpec((1, _RB, _W), lambda b, r: (b, r, 0)),
            pl.BlockSpec((1, _RB, _W), lambda b, r: (b, r, 0)),
        ],
        out_shape=[
            jax.ShapeDtypeStruct((_N, _H, _W), jnp.int32),
            jax.ShapeDtypeStruct((_N, _H, _W), jnp.float32),
        ],
    )(predict, target)


@functools.partial(
    pl.kernel,
    mesh=plsc.VectorSubcoreMesh(core_axis_name="c", subcore_axis_name="s"),
    out_type=jax.ShapeDtypeStruct((_NW, _B1), jnp.int32),
    scratch_types=[
        pltpu.VMEM((_SLAB,), jnp.int32),
        pltpu.VMEM((_B1,), jnp.int32),
    ],
    compiler_params=pltpu.CompilerParams(needs_layout_passes=False),
)
def _hist_pass_a(pred_hbm, out_hbm, buf, hist):
    wid = lax.axis_index("s") * 2 + lax.axis_index("c")
    base = wid * _PER_TILE

    @plsc.parallel_loop(0, _B1 // 16, unroll=8)
    def _zero(i):
        hist[pl.ds(i * 16, 16)] = jnp.zeros((16,), jnp.int32)

    ones = jnp.ones((16,), jnp.int32)

    def slab_body(s, c):
        pltpu.sync_copy(pred_hbm.at[pl.ds(base + s * _SLAB, _SLAB)], buf)

        @plsc.parallel_loop(0, _SLAB // 16, unroll=4)
        def _scatter(i):
            bits = buf[pl.ds(i * 16, 16)]
            ok = bits >= 0
            b = jnp.where(ok, bits >> 15, 0)
            plsc.addupdate_scatter(hist, [b], ones, mask=ok)

        return c

    lax.fori_loop(0, _PER_TILE // _SLAB, slab_body, 0)
    pltpu.sync_copy(hist, out_hbm.at[wid])


@functools.partial(
    pl.kernel,
    mesh=plsc.VectorSubcoreMesh(core_axis_name="c", subcore_axis_name="s"),
    out_type=jax.ShapeDtypeStruct((_NW, _B2), jnp.int32),
    scratch_types=[
        pltpu.VMEM((_SLAB,), jnp.int32),
        pltpu.VMEM((_B2,), jnp.int32),
        pltpu.VMEM((16,), jnp.int32),
    ],
    compiler_params=pltpu.CompilerParams(needs_layout_passes=False),
)
def _hist_pass_b(pred_hbm, b1_hbm, out_hbm, buf, hist, b1buf):
    wid = lax.axis_index("s") * 2 + lax.axis_index("c")
    base = wid * _PER_TILE
    pltpu.sync_copy(b1_hbm, b1buf)
    b1 = b1buf[...]

    @plsc.parallel_loop(0, _B2 // 16, unroll=8)
    def _zero(i):
        hist[pl.ds(i * 16, 16)] = jnp.zeros((16,), jnp.int32)

    ones = jnp.ones((16,), jnp.int32)

    def slab_body(s, c):
        pltpu.sync_copy(pred_hbm.at[pl.ds(base + s * _SLAB, _SLAB)], buf)

        @plsc.parallel_loop(0, _SLAB // 16, unroll=4)
        def _scatter(i):
            bits = buf[pl.ds(i * 16, 16)]
            ok = (bits >= 0) & ((bits >> 15) == b1)
            b = jnp.where(ok, bits & 0x7FFF, 0)
            plsc.addupdate_scatter(hist, [b], ones, mask=ok)

        return c

    lax.fori_loop(0, _PER_TILE // _SLAB, slab_body, 0)
    pltpu.sync_copy(hist, out_hbm.at[wid])


def _search(h, binidx, k, nbins, iters):
    def gcount(m):
        return jnp.sum(jnp.where(binidx > m, h, 0))

    def body(_, lohi):
        lo, hi = lohi
        mid = (lo + hi) // 2
        below = gcount(mid) < k
        return (jnp.where(below, lo, mid), jnp.where(below, mid, hi))

    lo, hi = lax.fori_loop(
        0, iters, body, (jnp.int32(-1), jnp.int32(nbins - 1))
    )
    return hi, gcount(hi)


def _scan_a_body(h_ref, b1_ref, kp_ref):
    h = jnp.sum(h_ref[...], axis=0)
    r = lax.broadcasted_iota(jnp.int32, h.shape, 0)
    c = lax.broadcasted_iota(jnp.int32, h.shape, 1)
    binidx = r * 128 + c
    nv = jnp.sum(h)
    k = jnp.floor(nv.astype(jnp.float32) * _RATIO).astype(jnp.int32)
    b1, above = _search(h, binidx, k, _B1, 18)
    b1_ref[...] = jnp.full((1, 128), b1, jnp.int32)
    kp_ref[...] = jnp.full((1, 128), k - above, jnp.int32)


def _scan_a_call(h1):
    return pl.pallas_call(
        _scan_a_body,
        out_shape=[
            jax.ShapeDtypeStruct((1, 128), jnp.int32),
            jax.ShapeDtypeStruct((1, 128), jnp.int32),
        ],
    )(h1)


def _scan_b_body(h_ref, b1_ref, kp_ref, t_ref):
    h = jnp.sum(h_ref[...], axis=0)
    r = lax.broadcasted_iota(jnp.int32, h.shape, 0)
    c = lax.broadcasted_iota(jnp.int32, h.shape, 1)
    binidx = r * 128 + c
    b1 = b1_ref[0, 0]
    kp = kp_ref[0, 0]
    b2, _ = _search(h, binidx, kp, _B2, 17)
    t_ref[...] = jnp.full((1, 128), (b1 << 15) | b2, jnp.int32)


def _scan_b_call(h2, b1v, kpv):
    return pl.pallas_call(
        _scan_b_body,
        in_specs=[
            pl.BlockSpec((_NW, _B2 // 128, 128), lambda: (0, 0, 0)),
            pl.BlockSpec(memory_space=pltpu.SMEM),
            pl.BlockSpec(memory_space=pltpu.SMEM),
        ],
        out_shape=jax.ShapeDtypeStruct((1, 128), jnp.int32),
    )(h2, b1v, kpv)


def _final_body(p_ref, n_ref, t_ref, out_ref, acc_ref):
    i = pl.program_id(0)
    j = pl.program_id(1)

    @pl.when((i == 0) & (j == 0))
    def _():
        acc_ref[0] = 0.0
        acc_ref[1] = 0.0

    p = p_ref[...]
    t = t_ref[0, 0]
    kept = (p >= 0) & (p <= t)
    acc_ref[0] += jnp.sum(jnp.where(kept, n_ref[...], 0.0))
    acc_ref[1] += jnp.sum(kept.astype(jnp.float32))

    @pl.when((i == _N - 1) & (j == _H // _RB - 1))
    def _():
        out_ref[...] = jnp.full(
            (1, 1), acc_ref[0] / jnp.maximum(acc_ref[1], 1.0), jnp.float32
        )


def _final_call(pred, nll, tv):
    grid = (_N, _H // _RB)
    return pl.pallas_call(
        _final_body,
        grid=grid,
        in_specs=[
            pl.BlockSpec((1, _RB, _W), lambda b, r: (b, r, 0)),
            pl.BlockSpec((1, _RB, _W), lambda b, r: (b, r, 0)),
            pl.BlockSpec(memory_space=pltpu.SMEM),
        ],
        out_specs=pl.BlockSpec((1, 1), lambda b, r: (0, 0)),
        out_shape=jax.ShapeDtypeStruct((1, 1), jnp.float32),
        scratch_shapes=[pltpu.SMEM((2,), jnp.float32)],
    )(pred, nll, tv)


def kernel(predict, target):
    pred, nll = _probs_call(predict, target)
    predf = pred.reshape(_NPIX)
    h1 = _hist_pass_a(predf)
    b1v, kpv = _scan_a_call(h1.reshape(_NW, _B1 // 128, 128))
    h2 = _hist_pass_b(predf, b1v[0, :16])
    tv = _scan_b_call(h2.reshape(_NW, _B2 // 128, 128), b1v, kpv)
    loss = _final_call(pred, nll, tv)
    return loss[0, 0]


# speedup vs baseline: 15.1950x; 1.0267x over previous
import functools

import jax
import jax.numpy as jnp
from jax import lax
from jax.experimental import pallas as pl
from jax.experimental.pallas import tpu as pltpu
from jax.experimental.pallas import tpu_sc as plsc

_IGNORE = 255
_RATIO = 0.7

_N, _C, _H, _W = 8, 19, 512, 512
_NPIX = _N * _H * _W
_RB = 32

_NW = 32
_PER_TILE = _NPIX // _NW
_SLAB = 32768
_B1 = 65536
_B2 = 32768


def _nll_body(x_ref, t_ref, w_ref):
    x = x_ref[...]
    tgt = t_ref[...]
    m = jnp.max(x, axis=1)
    se = jnp.sum(jnp.exp(x - m[:, None]), axis=1)
    cls = lax.broadcasted_iota(jnp.int32, x.shape, 1)
    xt = jnp.sum(jnp.where(cls == tgt[:, None], x, 0.0), axis=1)
    nll = jnp.log(se) - (xt - m)
    valid = tgt != _IGNORE
    w_ref[...] = lax.bitcast_convert_type(
        jnp.where(valid, nll, -1.0), jnp.int32
    )


def _nll_call(predict, target):
    grid = (_N, _H // _RB)
    return pl.pallas_call(
        _nll_body,
        grid=grid,
        in_specs=[
            pl.BlockSpec((1, _C, _RB, _W), lambda b, r: (b, 0, r, 0)),
            pl.BlockSpec((1, _RB, _W), lambda b, r: (b, r, 0)),
        ],
        out_specs=pl.BlockSpec((1, _RB, _W), lambda b, r: (b, r, 0)),
        out_shape=jax.ShapeDtypeStruct((_N, _H, _W), jnp.int32),
    )(predict, target)


@functools.partial(
    pl.kernel,
    mesh=plsc.VectorSubcoreMesh(core_axis_name="c", subcore_axis_name="s"),
    out_type=jax.ShapeDtypeStruct((_NW, _B1), jnp.int32),
    scratch_types=[
        pltpu.VMEM((_SLAB,), jnp.int32),
        pltpu.VMEM((_B1,), jnp.int32),
    ],
    compiler_params=pltpu.CompilerParams(needs_layout_passes=False),
)
def _hist_pass_a(bits_hbm, out_hbm, buf, hist):
    wid = lax.axis_index("s") * 2 + lax.axis_index("c")
    base = wid * _PER_TILE

    @plsc.parallel_loop(0, _B1 // 16, unroll=8)
    def _zero(i):
        hist[pl.ds(i * 16, 16)] = jnp.zeros((16,), jnp.int32)

    ones = jnp.ones((16,), jnp.int32)

    def slab_body(s, c):
        pltpu.sync_copy(bits_hbm.at[pl.ds(base + s * _SLAB, _SLAB)], buf)

        @plsc.parallel_loop(0, _SLAB // 16, unroll=4)
        def _scatter(i):
            bits = buf[pl.ds(i * 16, 16)]
            ok = bits >= 0
            b = jnp.where(ok, bits >> 15, 0)
            plsc.addupdate_scatter(hist, [b], ones, mask=ok)

        return c

    lax.fori_loop(0, _PER_TILE // _SLAB, slab_body, 0)
    pltpu.sync_copy(hist, out_hbm.at[wid])


@functools.partial(
    pl.kernel,
    mesh=plsc.VectorSubcoreMesh(core_axis_name="c", subcore_axis_name="s"),
    out_type=jax.ShapeDtypeStruct((_NW, _B2), jnp.int32),
    scratch_types=[
        pltpu.VMEM((_SLAB,), jnp.int32),
        pltpu.VMEM((_B2,), jnp.int32),
        pltpu.VMEM((16,), jnp.int32),
    ],
    compiler_params=pltpu.CompilerParams(needs_layout_passes=False),
)
def _hist_pass_b(bits_hbm, b1_hbm, out_hbm, buf, hist, b1buf):
    wid = lax.axis_index("s") * 2 + lax.axis_index("c")
    base = wid * _PER_TILE
    pltpu.sync_copy(b1_hbm, b1buf)
    b1 = b1buf[...]

    @plsc.parallel_loop(0, _B2 // 16, unroll=8)
    def _zero(i):
        hist[pl.ds(i * 16, 16)] = jnp.zeros((16,), jnp.int32)

    ones = jnp.ones((16,), jnp.int32)

    def slab_body(s, c):
        pltpu.sync_copy(bits_hbm.at[pl.ds(base + s * _SLAB, _SLAB)], buf)

        @plsc.parallel_loop(0, _SLAB // 16, unroll=4)
        def _scatter(i):
            bits = buf[pl.ds(i * 16, 16)]
            ok = (bits >= 0) & ((bits >> 15) == b1)
            b = jnp.where(ok, bits & 0x7FFF, 0)
            plsc.addupdate_scatter(hist, [b], ones, mask=ok)

        return c

    lax.fori_loop(0, _PER_TILE // _SLAB, slab_body, 0)
    pltpu.sync_copy(hist, out_hbm.at[wid])


def _search(h, binidx, k, nbins, iters):
    def lcount(m):
        return jnp.sum(jnp.where(binidx < m, h, 0))

    def body(_, lohi):
        lo, hi = lohi
        mid = (lo + hi) // 2
        above = lcount(mid + 1) >= k
        return (jnp.where(above, lo, mid), jnp.where(above, mid, hi))

    lo, hi = lax.fori_loop(
        0, iters, body, (jnp.int32(0), jnp.int32(nbins - 1))
    )
    return hi, lcount(hi)


def _scan_a_body(h_ref, b1_ref, kp_ref):
    h = jnp.sum(h_ref[...], axis=0)
    r = lax.broadcasted_iota(jnp.int32, h.shape, 0)
    c = lax.broadcasted_iota(jnp.int32, h.shape, 1)
    binidx = r * 128 + c
    nv = jnp.sum(h)
    k = jnp.floor(nv.astype(jnp.float32) * _RATIO).astype(jnp.int32)
    b1, below = _search(h, binidx, k, _B1, 17)
    b1_ref[...] = jnp.full((1, 128), b1, jnp.int32)
    kp_ref[...] = jnp.full((1, 128), k - below, jnp.int32)


def _scan_a_call(h1):
    return pl.pallas_call(
        _scan_a_body,
        out_shape=[
            jax.ShapeDtypeStruct((1, 128), jnp.int32),
            jax.ShapeDtypeStruct((1, 128), jnp.int32),
        ],
    )(h1)


def _final_body(h_ref, b1_ref, kp_ref, w_ref, out_ref, acc_ref):
    i = pl.program_id(0)
    j = pl.program_id(1)

    @pl.when((i == 0) & (j == 0))
    def _():
        h = jnp.sum(h_ref[...], axis=0)
        r = lax.broadcasted_iota(jnp.int32, h.shape, 0)
        c = lax.broadcasted_iota(jnp.int32, h.shape, 1)
        binidx = r * 128 + c
        b2, _ = _search(h, binidx, kp_ref[0, 0], _B2, 16)
        acc_ref[0] = 0.0
        acc_ref[1] = 0.0
        acc_ref[2] = lax.bitcast_convert_type(
            (b1_ref[0, 0] << 15) | b2, jnp.float32
        )

    w = w_ref[...]
    t = lax.bitcast_convert_type(acc_ref[2], jnp.int32)
    kept = w >= t
    acc_ref[0] += jnp.sum(
        jnp.where(kept, lax.bitcast_convert_type(w, jnp.float32), 0.0)
    )
    acc_ref[1] += jnp.sum(kept.astype(jnp.float32))

    @pl.when((i == _N - 1) & (j == _H // _RB - 1))
    def _():
        out_ref[...] = jnp.full(
            (1, 1), acc_ref[0] / jnp.maximum(acc_ref[1], 1.0), jnp.float32
        )


def _final_call(h2, b1v, kpv, w):
    grid = (_N, _H // _RB)
    return pl.pallas_call(
        _final_body,
        grid=grid,
        in_specs=[
            pl.BlockSpec((_NW, _B2 // 128, 128), lambda b, r: (0, 0, 0)),
            pl.BlockSpec(memory_space=pltpu.SMEM),
            pl.BlockSpec(memory_space=pltpu.SMEM),
            pl.BlockSpec((1, _RB, _W), lambda b, r: (b, r, 0)),
        ],
        out_specs=pl.BlockSpec((1, 1), lambda b, r: (0, 0)),
        out_shape=jax.ShapeDtypeStruct((1, 1), jnp.float32),
        scratch_shapes=[pltpu.SMEM((3,), jnp.float32)],
    )(h2, b1v, kpv, w)


def kernel(predict, target):
    w = _nll_call(predict, target)
    wf = w.reshape(_NPIX)
    h1 = _hist_pass_a(wf)
    b1v, kpv = _scan_a_call(h1.reshape(_NW, _B1 // 128, 128))
    h2 = _hist_pass_b(wf, b1v[0, :16])
    loss = _final_call(h2.reshape(_NW, _B2 // 128, 128), b1v, kpv, w)
    return loss[0, 0]


# speedup vs baseline: 27.1886x; 1.7893x over previous
import functools

import jax
import jax.numpy as jnp
from jax import lax
from jax.experimental import pallas as pl
from jax.experimental.pallas import tpu as pltpu
from jax.experimental.pallas import tpu_sc as plsc

_IGNORE = 255
_RATIO = 0.7

_N, _C, _H, _W = 8, 19, 512, 512
_NPIX = _N * _H * _W
_RB = 512

_NW = 32
_PER_TILE = _NPIX // _NW
_SLAB = 32768
_B1 = 65536
_B2 = 32768


def _nll_body(x_ref, t_ref, w_ref):
    x = x_ref[...]
    tgt = t_ref[...]
    m = jnp.max(x, axis=1)
    se = jnp.sum(jnp.exp(x - m[:, None]), axis=1)
    cls = lax.broadcasted_iota(jnp.int32, x.shape, 1)
    xt = jnp.sum(jnp.where(cls == tgt[:, None], x, 0.0), axis=1)
    nll = jnp.log(se) - (xt - m)
    valid = tgt != _IGNORE
    w_ref[...] = lax.bitcast_convert_type(
        jnp.where(valid, nll, -1.0), jnp.int32
    )


def _nll_call(predict, target):
    grid = (_N, _H // _RB)
    return pl.pallas_call(
        _nll_body,
        grid=grid,
        in_specs=[
            pl.BlockSpec((1, _C, _RB, _W), lambda b, r: (b, 0, r, 0)),
            pl.BlockSpec((1, _RB, _W), lambda b, r: (b, r, 0)),
        ],
        out_specs=pl.BlockSpec((1, _RB, _W), lambda b, r: (b, r, 0)),
        out_shape=jax.ShapeDtypeStruct((_N, _H, _W), jnp.int32),
    )(predict, target)


@functools.partial(
    pl.kernel,
    mesh=plsc.VectorSubcoreMesh(core_axis_name="c", subcore_axis_name="s"),
    out_type=jax.ShapeDtypeStruct((_NW, _B1), jnp.int32),
    scratch_types=[
        pltpu.VMEM((_SLAB,), jnp.int32),
        pltpu.VMEM((_B1,), jnp.int32),
    ],
    compiler_params=pltpu.CompilerParams(needs_layout_passes=False),
)
def _hist_pass_a(bits_hbm, out_hbm, buf, hist):
    wid = lax.axis_index("s") * 2 + lax.axis_index("c")
    base = wid * _PER_TILE

    @plsc.parallel_loop(0, _B1 // 16, unroll=8)
    def _zero(i):
        hist[pl.ds(i * 16, 16)] = jnp.zeros((16,), jnp.int32)

    ones = jnp.ones((16,), jnp.int32)

    def slab_body(s, c):
        pltpu.sync_copy(bits_hbm.at[pl.ds(base + s * _SLAB, _SLAB)], buf)

        @plsc.parallel_loop(0, _SLAB // 16, unroll=4)
        def _scatter(i):
            bits = buf[pl.ds(i * 16, 16)]
            ok = bits >= 0
            b = jnp.where(ok, bits >> 15, 0)
            plsc.addupdate_scatter(hist, [b], ones, mask=ok)

        return c

    lax.fori_loop(0, _PER_TILE // _SLAB, slab_body, 0)
    pltpu.sync_copy(hist, out_hbm.at[wid])


@functools.partial(
    pl.kernel,
    mesh=plsc.VectorSubcoreMesh(core_axis_name="c", subcore_axis_name="s"),
    out_type=jax.ShapeDtypeStruct((_NW, _B2), jnp.int32),
    scratch_types=[
        pltpu.VMEM((_SLAB,), jnp.int32),
        pltpu.VMEM((_B2,), jnp.int32),
        pltpu.VMEM((16,), jnp.int32),
    ],
    compiler_params=pltpu.CompilerParams(needs_layout_passes=False),
)
def _hist_pass_b(bits_hbm, b1_hbm, out_hbm, buf, hist, b1buf):
    wid = lax.axis_index("s") * 2 + lax.axis_index("c")
    base = wid * _PER_TILE
    pltpu.sync_copy(b1_hbm, b1buf)
    b1 = b1buf[...]

    @plsc.parallel_loop(0, _B2 // 16, unroll=8)
    def _zero(i):
        hist[pl.ds(i * 16, 16)] = jnp.zeros((16,), jnp.int32)

    ones = jnp.ones((16,), jnp.int32)

    def slab_body(s, c):
        pltpu.sync_copy(bits_hbm.at[pl.ds(base + s * _SLAB, _SLAB)], buf)

        @plsc.parallel_loop(0, _SLAB // 16, unroll=4)
        def _scatter(i):
            bits = buf[pl.ds(i * 16, 16)]
            ok = (bits >= 0) & ((bits >> 15) == b1)
            b = jnp.where(ok, bits & 0x7FFF, 0)
            plsc.addupdate_scatter(hist, [b], ones, mask=ok)

        return c

    lax.fori_loop(0, _PER_TILE // _SLAB, slab_body, 0)
    pltpu.sync_copy(hist, out_hbm.at[wid])


def _search(h, binidx, k, nbins, iters):
    def lcount(m):
        return jnp.sum(jnp.where(binidx < m, h, 0))

    def body(_, lohi):
        lo, hi = lohi
        mid = (lo + hi) // 2
        above = lcount(mid + 1) >= k
        return (jnp.where(above, lo, mid), jnp.where(above, mid, hi))

    lo, hi = lax.fori_loop(
        0, iters, body, (jnp.int32(0), jnp.int32(nbins - 1))
    )
    return hi, lcount(hi)


def _scan_a_body(h_ref, b1_ref, kp_ref):
    h = jnp.sum(h_ref[...], axis=0)
    r = lax.broadcasted_iota(jnp.int32, h.shape, 0)
    c = lax.broadcasted_iota(jnp.int32, h.shape, 1)
    binidx = r * 128 + c
    nv = jnp.sum(h)
    k = jnp.floor(nv.astype(jnp.float32) * _RATIO).astype(jnp.int32)
    b1, below = _search(h, binidx, k, _B1, 17)
    b1_ref[...] = jnp.full((1, 128), b1, jnp.int32)
    kp_ref[...] = jnp.full((1, 128), k - below, jnp.int32)


def _scan_a_call(h1):
    return pl.pallas_call(
        _scan_a_body,
        out_shape=[
            jax.ShapeDtypeStruct((1, 128), jnp.int32),
            jax.ShapeDtypeStruct((1, 128), jnp.int32),
        ],
    )(h1)


def _final_body(h_ref, b1_ref, kp_ref, w_ref, out_ref):
    h = jnp.sum(h_ref[...], axis=0)
    r = lax.broadcasted_iota(jnp.int32, h.shape, 0)
    c = lax.broadcasted_iota(jnp.int32, h.shape, 1)
    binidx = r * 128 + c
    b2, _ = _search(h, binidx, kp_ref[0, 0], _B2, 16)
    t = (b1_ref[0, 0] << 15) | b2

    w = w_ref[...]
    kept = w >= t
    num = jnp.sum(
        jnp.where(kept, lax.bitcast_convert_type(w, jnp.float32), 0.0)
    )
    den = jnp.sum(kept.astype(jnp.float32))
    out_ref[...] = jnp.full((1, 1), num / jnp.maximum(den, 1.0), jnp.float32)


def _final_call(h2, b1v, kpv, w):
    return pl.pallas_call(
        _final_body,
        in_specs=[
            pl.BlockSpec((_NW, _B2 // 128, 128), lambda: (0, 0, 0)),
            pl.BlockSpec(memory_space=pltpu.SMEM),
            pl.BlockSpec(memory_space=pltpu.SMEM),
            pl.BlockSpec((_N, _H, _W), lambda: (0, 0, 0)),
        ],
        out_specs=pl.BlockSpec((1, 1), lambda: (0, 0)),
        out_shape=jax.ShapeDtypeStruct((1, 1), jnp.float32),
    )(h2, b1v, kpv, w)


def kernel(predict, target):
    w = _nll_call(predict, target)
    wf = w.reshape(_NPIX)
    h1 = _hist_pass_a(wf)
    b1v, kpv = _scan_a_call(h1.reshape(_NW, _B1 // 128, 128))
    h2 = _hist_pass_b(wf, b1v[0, :16])
    loss = _final_call(h2.reshape(_NW, _B2 // 128, 128), b1v, kpv, w)
    return loss[0, 0]
